# baseline (speedup 1.0000x reference)
import functools

import jax
import jax.numpy as jnp
from jax import lax
from jax.experimental import pallas as pl
from jax.experimental.pallas import tpu as pltpu
from jax.experimental.pallas import tpu_sc as plsc

VOCAB = 1000000
EMBED = 64
BATCH = 16384
CTX = 5

NC, NS, L = 2, 16, 16
NW = NC * NS
PER_W = BATCH // NW
CB = 16
NCHUNK = PER_W // CB
NLK_C = CB * CTX
NPAIR = CB * CTX
G = VOCAB // L
TROWS = CB * EMBED
CROWS = NLK_C * EMBED


def _gen_indices(ids, idx_buf, lk0, iota):
    g = jnp.right_shift(ids, 4)
    par = jnp.bitwise_and(ids, 15)
    pos0 = (iota + lk0) * EMBED

    def estep(i, carry):
        val, pos = carry
        for _ in range(4):
            plsc.store_scatter(idx_buf, [pos], val)
            val = val + G
            pos = pos + 1
        return val, pos

    lax.fori_loop(0, EMBED // 4, estep, (g, pos0))
    return par


def _body(ttab, ctab, tidx, cidx, out,
          stage_t, stage_c, par_t, par_c, t_idx, c_idx, t_blk, c_blk, out_v, sem):
    wid = lax.axis_index("s") * NC + lax.axis_index("c")
    iota = lax.iota(jnp.int32, L)

    def chunk(ch, _):
        b0 = wid * PER_W + ch * CB
        pltpu.sync_copy(tidx.at[pl.ds(b0, CB)], stage_t)
        pltpu.sync_copy(cidx.at[pl.ds(b0 * CTX, NLK_C)], stage_c)

        par_t[...] = _gen_indices(stage_t[...], t_idx, 0, iota)
        for q in range(CTX):
            par_c[pl.ds(q * L, L)] = _gen_indices(
                stage_c[pl.ds(q * L, L)], c_idx, q * L, iota)

        cps = []
        for j in range(TROWS // 128):
            cps.append(pltpu.async_copy(
                ttab.at[t_idx.at[pl.ds(j * 128, 128)]],
                t_blk.at[pl.ds(j * 128, 128)], sem))
        for j in range(CROWS // 128):
            cps.append(pltpu.async_copy(
                ctab.at[c_idx.at[pl.ds(j * 128, 128)]],
                c_blk.at[pl.ds(j * 128, 128)], sem))
        for cp in cps:
            cp.wait()

        for p0 in range(0, NPAIR, L):
            p = iota + p0
            b_loc = jnp.right_shift(p * 13108, 16)
            pt = plsc.load_gather(par_t, [b_loc])
            pc = par_c[pl.ds(p0, L)]
            zero = jnp.zeros((L,), jnp.float32)

            def estep(i, carry):
                acc, t_row, c_row = carry
                for _ in range(4):
                    tv = plsc.load_gather(t_blk, [t_row, pt])
                    cv = plsc.load_gather(c_blk, [c_row, pc])
                    acc = acc + tv * cv
                    t_row = t_row + 1
                    c_row = c_row + 1
                return acc, t_row, c_row

            acc, _, _ = lax.fori_loop(
                0, EMBED // 4, estep, (zero, b_loc * EMBED, p * EMBED))
            out_v[pl.ds(p0, L)] = acc

        pltpu.sync_copy(out_v, out.at[pl.ds(b0 * CTX, NPAIR)])
        return 0

    lax.fori_loop(0, NCHUNK, chunk, 0)


@jax.jit
def kernel(target, context, target_table, context_table):
    tt = target_table.T.reshape(EMBED * G, L)
    ct = context_table.T.reshape(EMBED * G, L)
    tidx = target.astype(jnp.int32)
    cidx = context.astype(jnp.int32).reshape(BATCH * CTX)
    fn = pl.kernel(
        _body,
        out_type=jax.ShapeDtypeStruct((BATCH * CTX,), jnp.float32),
        mesh=plsc.VectorSubcoreMesh(core_axis_name="c", subcore_axis_name="s"),
        compiler_params=pltpu.CompilerParams(
            needs_layout_passes=False, use_tc_tiling_on_sc=False),
        scratch_types=[
            pltpu.VMEM((CB,), jnp.int32),
            pltpu.VMEM((NLK_C,), jnp.int32),
            pltpu.VMEM((CB,), jnp.int32),
            pltpu.VMEM((NLK_C,), jnp.int32),
            pltpu.VMEM((TROWS,), jnp.int32),
            pltpu.VMEM((CROWS,), jnp.int32),
            pltpu.VMEM((TROWS, L), jnp.float32),
            pltpu.VMEM((CROWS, L), jnp.float32),
            pltpu.VMEM((NPAIR,), jnp.float32),
            pltpu.SemaphoreType.DMA,
        ],
    )
    dots = fn(tt, ct, tidx, cidx)
    return dots.reshape(BATCH, CTX)


# speedup vs baseline: 10.1683x; 10.1683x over previous
import functools

import jax
import jax.numpy as jnp
from jax import lax
from jax.experimental import pallas as pl
from jax.experimental.pallas import tpu as pltpu
from jax.experimental.pallas import tpu_sc as plsc

VOCAB = 1000000
EMBED = 64
BATCH = 16384
CTX = 5

NC, NS, L = 2, 16, 16
NW = NC * NS
PER_W = BATCH // NW
CB = 128
CHUNKS = PER_W // CB
NLK_C = CB * CTX
PACKED = EMBED // 2
HI = -65536

VBLK = 2048
GRID = (VOCAB + VBLK - 1) // VBLK
RB = VBLK // 4
ROWS = GRID * RB


def _pack_body(src, dst):
    x = src[...]
    lo = lax.convert_element_type(x[:PACKED, :], jnp.bfloat16)
    hi = lax.convert_element_type(x[PACKED:, :], jnp.bfloat16)
    lo_u = lax.convert_element_type(
        lax.bitcast_convert_type(lo, jnp.uint16), jnp.uint32)
    hi_u = lax.convert_element_type(
        lax.bitcast_convert_type(hi, jnp.uint16), jnp.uint32)
    pk = lax.bitcast_convert_type(
        jnp.bitwise_or(lo_u, jnp.left_shift(hi_u, 16)),
        jnp.int32)
    pt = jnp.transpose(pk, (1, 0))
    dst[...] = jnp.concatenate(
        [pt[q * RB:(q + 1) * RB, :] for q in range(4)], axis=1)


def _pack_table(table):
    return pl.pallas_call(
        _pack_body,
        grid=(GRID,),
        in_specs=[pl.BlockSpec((EMBED, VBLK), lambda i: (0, i))],
        out_specs=pl.BlockSpec((RB, 4 * PACKED), lambda i: (i, 0)),
        out_shape=jax.ShapeDtypeStruct((ROWS, 4 * PACKED), jnp.int32),
    )(table.T)


def _body(ttab, ctab, tidx, cidx, out,
          t_idx, c_idx, t_pos0, c_pos0, t_blk, c_blk, out_v, sem):
    wid = lax.axis_index("s") * NC + lax.axis_index("c")
    iota = lax.iota(jnp.int32, L)

    def chunk(ch, _):
        b0 = wid * PER_W + ch * CB
        pltpu.sync_copy(tidx.at[pl.ds(b0, CB)], t_idx)
        pltpu.sync_copy(cidx.at[pl.ds(b0 * CTX, NLK_C)], c_idx)

        def prep_t(g, _):
            v = t_idx[pl.ds(g * L, L)]
            t_pos0[pl.ds(g * L, L)] = (
                jnp.bitwise_and(jnp.right_shift(v, 9), 3) * PACKED)
            t_idx[pl.ds(g * L, L)] = jnp.bitwise_or(
                jnp.left_shift(jnp.right_shift(v, 11), 9),
                jnp.bitwise_and(v, 511))
            return 0

        def prep_c(g, _):
            v = c_idx[pl.ds(g * L, L)]
            c_pos0[pl.ds(g * L, L)] = (
                jnp.bitwise_and(jnp.right_shift(v, 9), 3) * PACKED)
            c_idx[pl.ds(g * L, L)] = jnp.bitwise_or(
                jnp.left_shift(jnp.right_shift(v, 11), 9),
                jnp.bitwise_and(v, 511))
            return 0

        lax.fori_loop(0, CB // L, prep_t, 0)
        lax.fori_loop(0, NLK_C // L, prep_c, 0)

        cps = [pltpu.async_copy(ttab.at[t_idx], t_blk, sem)]
        for j in range(CTX):
            cps.append(pltpu.async_copy(
                ctab.at[c_idx.at[pl.ds(j * CB, CB)]],
                c_blk.at[pl.ds(j * CB, CB)], sem))
        for cp in cps:
            cp.wait()

        def pair_group(grp, _):
            p0 = grp * L
            p = iota + p0
            b_loc = jnp.right_shift(p * 52429, 18)
            t_base = plsc.load_gather(t_pos0, [b_loc])
            c_base = c_pos0[pl.ds(p0, L)]
            acc = jnp.zeros((L,), jnp.float32)
            for k in range(PACKED):
                tx = plsc.load_gather(t_blk, [b_loc, t_base + k])
                cx = plsc.load_gather(c_blk, [p, c_base + k])
                tlo = plsc.bitcast(jnp.left_shift(tx, 16), jnp.float32)
                thi = plsc.bitcast(jnp.bitwise_and(tx, HI), jnp.float32)
                clo = plsc.bitcast(jnp.left_shift(cx, 16), jnp.float32)
                chi = plsc.bitcast(jnp.bitwise_and(cx, HI), jnp.float32)
                acc = acc + tlo * clo + thi * chi
            out_v[pl.ds(p0, L)] = acc
            return 0

        lax.fori_loop(0, NLK_C // L, pair_group, 0)
        pltpu.sync_copy(out_v, out.at[pl.ds(b0 * CTX, NLK_C)])
        return 0

    lax.fori_loop(0, CHUNKS, chunk, 0)


@jax.jit
def kernel(target, context, target_table, context_table):
    tt = _pack_table(target_table)
    ct = _pack_table(context_table)
    tidx = target.astype(jnp.int32)
    cidx = context.astype(jnp.int32).reshape(BATCH * CTX)
    fn = pl.kernel(
        _body,
        out_type=jax.ShapeDtypeStruct((BATCH * CTX,), jnp.float32),
        mesh=plsc.VectorSubcoreMesh(core_axis_name="c", subcore_axis_name="s"),
        compiler_params=pltpu.CompilerParams(needs_layout_passes=False),
        scratch_types=[
            pltpu.VMEM((CB,), jnp.int32),
            pltpu.VMEM((NLK_C,), jnp.int32),
            pltpu.VMEM((CB,), jnp.int32),
            pltpu.VMEM((NLK_C,), jnp.int32),
            pltpu.VMEM((CB, 4 * PACKED), jnp.int32),
            pltpu.VMEM((NLK_C, 4 * PACKED), jnp.int32),
            pltpu.VMEM((NLK_C,), jnp.float32),
            pltpu.SemaphoreType.DMA,
        ],
    )
    dots = fn(tt, ct, tidx, cidx)
    return dots.reshape(BATCH, CTX)


# speedup vs baseline: 12.5122x; 1.2305x over previous
import functools

import jax
import jax.numpy as jnp
from jax import lax
from jax.experimental import pallas as pl
from jax.experimental.pallas import tpu as pltpu
from jax.experimental.pallas import tpu_sc as plsc

VOCAB = 1000000
EMBED = 64
BATCH = 16384
CTX = 5

NC, NS, L = 2, 16, 16
NW = NC * NS
PER_W = BATCH // NW
CB = 128
CHUNKS = PER_W // CB
NLK_C = CB * CTX
PACKED = EMBED // 2
HI = -65536

VBLK = 2048
GRID = (VOCAB + VBLK - 1) // VBLK
RB = VBLK // 4
ROWS = GRID * RB


def _pack_body(src, dst):
    x = src[...]
    parts = []
    for q in range(4):
        xq = x[:, q * RB:(q + 1) * RB]
        lo = lax.convert_element_type(xq[:PACKED, :], jnp.bfloat16)
        hi = lax.convert_element_type(xq[PACKED:, :], jnp.bfloat16)
        lo_u = lax.convert_element_type(
            lax.bitcast_convert_type(lo, jnp.uint16), jnp.uint32)
        hi_u = lax.convert_element_type(
            lax.bitcast_convert_type(hi, jnp.uint16), jnp.uint32)
        parts.append(jnp.bitwise_or(lo_u, jnp.left_shift(hi_u, 16)))
    pk2 = lax.bitcast_convert_type(
        jnp.concatenate(parts, axis=0), jnp.int32)
    dst[...] = jnp.transpose(pk2, (1, 0))


def _pack_table(table):
    return pl.pallas_call(
        _pack_body,
        grid=(GRID,),
        in_specs=[pl.BlockSpec((EMBED, VBLK), lambda i: (0, i))],
        out_specs=pl.BlockSpec((RB, 4 * PACKED), lambda i: (i, 0)),
        out_shape=jax.ShapeDtypeStruct((ROWS, 4 * PACKED), jnp.int32),
    )(table.T)


def _body(ttab, ctab, tidx, cidx, out,
          t_idx, c_idx, t_pos0, c_pos0, t_blk, c_blk, out_v, sem):
    wid = lax.axis_index("s") * NC + lax.axis_index("c")
    iota = lax.iota(jnp.int32, L)

    def chunk(ch, _):
        b0 = wid * PER_W + ch * CB
        pltpu.sync_copy(tidx.at[pl.ds(b0, CB)], t_idx)
        pltpu.sync_copy(cidx.at[pl.ds(b0 * CTX, NLK_C)], c_idx)

        def prep_t(g, _):
            v = t_idx[pl.ds(g * L, L)]
            t_pos0[pl.ds(g * L, L)] = (
                jnp.bitwise_and(jnp.right_shift(v, 9), 3) * PACKED)
            t_idx[pl.ds(g * L, L)] = jnp.bitwise_or(
                jnp.left_shift(jnp.right_shift(v, 11), 9),
                jnp.bitwise_and(v, 511))
            return 0

        def prep_c(g, _):
            v = c_idx[pl.ds(g * L, L)]
            c_pos0[pl.ds(g * L, L)] = (
                jnp.bitwise_and(jnp.right_shift(v, 9), 3) * PACKED)
            c_idx[pl.ds(g * L, L)] = jnp.bitwise_or(
                jnp.left_shift(jnp.right_shift(v, 11), 9),
                jnp.bitwise_and(v, 511))
            return 0

        lax.fori_loop(0, CB // L, prep_t, 0)
        lax.fori_loop(0, NLK_C // L, prep_c, 0)

        cps = [pltpu.async_copy(ttab.at[t_idx], t_blk, sem)]
        for j in range(CTX):
            cps.append(pltpu.async_copy(
                ctab.at[c_idx.at[pl.ds(j * CB, CB)]],
                c_blk.at[pl.ds(j * CB, CB)], sem))
        for cp in cps:
            cp.wait()

        def pair_group(grp, _):
            p0 = grp * L
            p = iota + p0
            b_loc = jnp.right_shift(p * 52429, 18)
            t_base = plsc.load_gather(t_pos0, [b_loc])
            c_base = c_pos0[pl.ds(p0, L)]
            acc = jnp.zeros((L,), jnp.float32)
            for k in range(PACKED):
                tx = plsc.load_gather(t_blk, [b_loc, t_base + k])
                cx = plsc.load_gather(c_blk, [p, c_base + k])
                tlo = plsc.bitcast(jnp.left_shift(tx, 16), jnp.float32)
                thi = plsc.bitcast(jnp.bitwise_and(tx, HI), jnp.float32)
                clo = plsc.bitcast(jnp.left_shift(cx, 16), jnp.float32)
                chi = plsc.bitcast(jnp.bitwise_and(cx, HI), jnp.float32)
                acc = acc + tlo * clo + thi * chi
            out_v[pl.ds(p0, L)] = acc
            return 0

        lax.fori_loop(0, NLK_C // L, pair_group, 0)
        pltpu.sync_copy(out_v, out.at[pl.ds(b0 * CTX, NLK_C)])
        return 0

    lax.fori_loop(0, CHUNKS, chunk, 0)


@jax.jit
def kernel(target, context, target_table, context_table):
    tt = _pack_table(target_table)
    ct = _pack_table(context_table)
    tidx = target.astype(jnp.int32)
    cidx = context.astype(jnp.int32).reshape(BATCH * CTX)
    fn = pl.kernel(
        _body,
        out_type=jax.ShapeDtypeStruct((BATCH * CTX,), jnp.float32),
        mesh=plsc.VectorSubcoreMesh(core_axis_name="c", subcore_axis_name="s"),
        compiler_params=pltpu.CompilerParams(needs_layout_passes=False),
        scratch_types=[
            pltpu.VMEM((CB,), jnp.int32),
            pltpu.VMEM((NLK_C,), jnp.int32),
            pltpu.VMEM((CB,), jnp.int32),
            pltpu.VMEM((NLK_C,), jnp.int32),
            pltpu.VMEM((CB, 4 * PACKED), jnp.int32),
            pltpu.VMEM((NLK_C, 4 * PACKED), jnp.int32),
            pltpu.VMEM((NLK_C,), jnp.float32),
            pltpu.SemaphoreType.DMA,
        ],
    )
    dots = fn(tt, ct, tidx, cidx)
    return dots.reshape(BATCH, CTX)


# speedup vs baseline: 22.6382x; 1.8093x over previous
import functools

import jax
import jax.numpy as jnp
from jax import lax
from jax.experimental import pallas as pl
from jax.experimental.pallas import tpu as pltpu
from jax.experimental.pallas import tpu_sc as plsc

VOCAB = 1000000
EMBED = 64
BATCH = 16384
CTX = 5

NC, NS, L = 2, 16, 16
NW = NC * NS
PER_W = BATCH // NW
CB = 128
CHUNKS = PER_W // CB
NLK_C = CB * CTX
PACKED = EMBED // 2
HI = -65536

VBLK = 8192
GRID = (VOCAB + VBLK - 1) // VBLK
RB = VBLK // 4
ROWS = GRID * RB
SB = 13
SR = 11


def _pack_body(src, dst):
    x = src[...]
    parts = []
    for q in range(4):
        xq = x[:, q * RB:(q + 1) * RB]
        lo = lax.convert_element_type(xq[:PACKED, :], jnp.bfloat16)
        hi = lax.convert_element_type(xq[PACKED:, :], jnp.bfloat16)
        lo_u = lax.convert_element_type(
            lax.bitcast_convert_type(lo, jnp.uint16), jnp.uint32)
        hi_u = lax.convert_element_type(
            lax.bitcast_convert_type(hi, jnp.uint16), jnp.uint32)
        parts.append(jnp.bitwise_or(lo_u, jnp.left_shift(hi_u, 16)))
    pk2 = lax.bitcast_convert_type(
        jnp.concatenate(parts, axis=0), jnp.int32)
    dst[...] = jnp.transpose(pk2, (1, 0))


def _pack_table(table):
    return pl.pallas_call(
        _pack_body,
        grid=(GRID,),
        in_specs=[pl.BlockSpec((EMBED, VBLK), lambda i: (0, i))],
        out_specs=pl.BlockSpec((RB, 4 * PACKED), lambda i: (i, 0)),
        out_shape=jax.ShapeDtypeStruct((ROWS, 4 * PACKED), jnp.int32),
    )(table.T)


def _body(ttab, ctab, tidx, cidx, out,
          t_idx, c_idx, t_pos0, c_pos0, t_blk, c_blk, out_v, sem):
    wid = lax.axis_index("s") * NC + lax.axis_index("c")
    iota = lax.iota(jnp.int32, L)

    def chunk(ch, _):
        b0 = wid * PER_W + ch * CB
        pltpu.sync_copy(tidx.at[pl.ds(b0, CB)], t_idx)
        pltpu.sync_copy(cidx.at[pl.ds(b0 * CTX, NLK_C)], c_idx)

        def prep_t(g, _):
            v = t_idx[pl.ds(g * L, L)]
            t_pos0[pl.ds(g * L, L)] = (
                jnp.bitwise_and(jnp.right_shift(v, SR), 3) * PACKED)
            t_idx[pl.ds(g * L, L)] = jnp.bitwise_or(
                jnp.left_shift(jnp.right_shift(v, SB), SR),
                jnp.bitwise_and(v, RB - 1))
            return 0

        def prep_c(g, _):
            v = c_idx[pl.ds(g * L, L)]
            c_pos0[pl.ds(g * L, L)] = (
                jnp.bitwise_and(jnp.right_shift(v, SR), 3) * PACKED)
            c_idx[pl.ds(g * L, L)] = jnp.bitwise_or(
                jnp.left_shift(jnp.right_shift(v, SB), SR),
                jnp.bitwise_and(v, RB - 1))
            return 0

        lax.fori_loop(0, CB // L, prep_t, 0)
        lax.fori_loop(0, NLK_C // L, prep_c, 0)

        cps = [pltpu.async_copy(ttab.at[t_idx], t_blk, sem)]
        for j in range(CTX):
            cps.append(pltpu.async_copy(
                ctab.at[c_idx.at[pl.ds(j * CB, CB)]],
                c_blk.at[pl.ds(j * CB, CB)], sem))
        for cp in cps:
            cp.wait()

        def pair_group(grp, _):
            p0 = grp * L
            p = iota + p0
            b_loc = jnp.right_shift(p * 52429, 18)
            t_base = plsc.load_gather(t_pos0, [b_loc])
            c_base = c_pos0[pl.ds(p0, L)]
            acc = jnp.zeros((L,), jnp.float32)
            for k in range(PACKED):
                tx = plsc.load_gather(t_blk, [b_loc, t_base + k])
                cx = plsc.load_gather(c_blk, [p, c_base + k])
                tlo = plsc.bitcast(jnp.left_shift(tx, 16), jnp.float32)
                thi = plsc.bitcast(jnp.bitwise_and(tx, HI), jnp.float32)
                clo = plsc.bitcast(jnp.left_shift(cx, 16), jnp.float32)
                chi = plsc.bitcast(jnp.bitwise_and(cx, HI), jnp.float32)
                acc = acc + tlo * clo + thi * chi
            out_v[pl.ds(p0, L)] = acc
            return 0

        lax.fori_loop(0, NLK_C // L, pair_group, 0)
        pltpu.sync_copy(out_v, out.at[pl.ds(b0 * CTX, NLK_C)])
        return 0

    lax.fori_loop(0, CHUNKS, chunk, 0)


@jax.jit
def kernel(target, context, target_table, context_table):
    tt = _pack_table(target_table)
    ct = _pack_table(context_table)
    tidx = target.astype(jnp.int32)
    cidx = context.astype(jnp.int32).reshape(BATCH * CTX)
    fn = pl.kernel(
        _body,
        out_type=jax.ShapeDtypeStruct((BATCH * CTX,), jnp.float32),
        mesh=plsc.VectorSubcoreMesh(core_axis_name="c", subcore_axis_name="s"),
        compiler_params=pltpu.CompilerParams(needs_layout_passes=False),
        scratch_types=[
            pltpu.VMEM((CB,), jnp.int32),
            pltpu.VMEM((NLK_C,), jnp.int32),
            pltpu.VMEM((CB,), jnp.int32),
            pltpu.VMEM((NLK_C,), jnp.int32),
            pltpu.VMEM((CB, 4 * PACKED), jnp.int32),
            pltpu.VMEM((NLK_C, 4 * PACKED), jnp.int32),
            pltpu.VMEM((NLK_C,), jnp.float32),
            pltpu.SemaphoreType.DMA,
        ],
    )
    dots = fn(tt, ct, tidx, cidx)
    return dots.reshape(BATCH, CTX)
